# Initial kernel scaffold; baseline (speedup 1.0000x reference)
import jax
import jax.numpy as jnp
from jax.experimental import pallas as pl


def kernel(x, edge_index, batch, head_W, head_b, head_g, head_be, gin_W, gin_b, gin_g, gin_be, lin_W, lin_b, tail_W, tail_b):
    raise NotImplementedError("write your pallas kernel here")


# speedup vs baseline: 5.9713x; 5.9713x over previous
import functools

import jax
import jax.numpy as jnp
from jax import lax
from jax.experimental import pallas as pl
from jax.experimental.pallas import tpu as pltpu
from jax.experimental.pallas import tpu_sc as plsc

N = 10000
E = 320000
DF = 128
H = 64
T = 4

NC = 2
NS = 16
NW = NC * NS
C = 128
CH = -(-E // (NW * C))
E_PAD = NW * CH * C
NPAD = N + 8
NA = 10240
RPT = NA // NS

_EPS = 1e-5


def _bn_relu(y, g, b):
    mean = jnp.mean(y, axis=0, keepdims=True)
    var = jnp.mean((y - mean) ** 2, axis=0, keepdims=True)
    return jnp.maximum((y - mean) / jnp.sqrt(var + _EPS) * g + b, 0.0)


def _head_body(x_ref, w_ref, b_ref, g_ref, be_ref, out_ref):
    y = jnp.dot(x_ref[...], w_ref[...], preferred_element_type=jnp.float32)
    h = _bn_relu(y + b_ref[...], g_ref[...], be_ref[...])
    out_ref[...] = jnp.concatenate(
        [h, jnp.zeros((NPAD - N, H), jnp.float32)], axis=0)


def _tail_body(h_ref, p_ref, gw_ref, gb_ref, gg_ref, gbe_ref,
               lw_ref, lb_ref, tw_ref, tb_ref, out_ref):
    z = h_ref[0:N, :] + p_ref[0, 0:N, :] + p_ref[1, 0:N, :]
    y = jnp.dot(z, gw_ref[...], preferred_element_type=jnp.float32)
    h2 = _bn_relu(y + gb_ref[...], gg_ref[...], gbe_ref[...])
    w2 = jnp.dot(lw_ref[...], tw_ref[...], preferred_element_type=jnp.float32)
    b2 = jnp.dot(lb_ref[...], tw_ref[...],
                 preferred_element_type=jnp.float32) + tb_ref[...]
    out_ref[...] = jnp.dot(h2, w2, preferred_element_type=jnp.float32) + b2


def _seg_sum_body(h_hbm, src_hbm, dst_hbm, zeros_hbm, out_hbm,
                  src_v, dst_v, rows_v, aggr_s, sem):
    cid = lax.axis_index("c")
    sid = lax.axis_index("s")
    wid = cid * NS + sid
    pltpu.sync_copy(zeros_hbm.at[pl.ds(sid * RPT, RPT)],
                    aggr_s.at[pl.ds(sid * RPT, RPT)])
    pltpu.sync_copy(src_hbm.at[wid], src_v)
    pltpu.sync_copy(dst_hbm.at[wid], dst_v)
    plsc.subcore_barrier()

    @pl.loop(0, CH)
    def _(j):
        pltpu.async_copy(h_hbm.at[src_v.at[j]], rows_v, sem).wait()
        pltpu.sync_copy(rows_v, aggr_s.at[dst_v.at[j]], add=True)

    plsc.subcore_barrier()
    pltpu.sync_copy(aggr_s.at[pl.ds(sid * RPT, RPT)],
                    out_hbm.at[cid, pl.ds(sid * RPT, RPT)])


_seg_sum = functools.partial(
    pl.kernel,
    out_type=jax.ShapeDtypeStruct((NC, NA, H), jnp.float32),
    mesh=plsc.VectorSubcoreMesh(core_axis_name="c", subcore_axis_name="s"),
    scratch_types=[
        pltpu.VMEM((CH, C), jnp.int32),
        pltpu.VMEM((CH, C), jnp.int32),
        pltpu.VMEM((C, H), jnp.float32),
        pltpu.VMEM_SHARED((NA, H), jnp.float32),
        pltpu.SemaphoreType.DMA,
    ],
    compiler_params=pltpu.CompilerParams(use_tc_tiling_on_sc=False),
)(_seg_sum_body)


def kernel(x, edge_index, batch, head_W, head_b, head_g, head_be,
           gin_W, gin_b, gin_g, gin_be, lin_W, lin_b, tail_W, tail_b):
    del batch
    h_pad = pl.pallas_call(
        _head_body,
        out_shape=jax.ShapeDtypeStruct((NPAD, H), jnp.float32),
    )(x, head_W, head_b.reshape(1, H), head_g.reshape(1, H),
      head_be.reshape(1, H))

    src = jnp.concatenate(
        [edge_index[0], jnp.full((E_PAD - E,), N, jnp.int32)]).reshape(NW, CH, C)
    dst = jnp.concatenate(
        [edge_index[1], jnp.zeros((E_PAD - E,), jnp.int32)]).reshape(NW, CH, C)
    zeros = jnp.zeros((NA, H), jnp.float32)
    parts = _seg_sum(h_pad, src, dst, zeros)

    out = pl.pallas_call(
        _tail_body,
        out_shape=jax.ShapeDtypeStruct((N, T), jnp.float32),
    )(h_pad, parts, gin_W, gin_b.reshape(1, H), gin_g.reshape(1, H),
      gin_be.reshape(1, H), lin_W, lin_b.reshape(1, H), tail_W,
      tail_b.reshape(1, T))
    return out
